# baseline (speedup 1.0000x reference)
import functools

import jax
import jax.numpy as jnp
from jax import lax
from jax.experimental import pallas as pl
from jax.experimental.pallas import tpu as pltpu
from jax.experimental.pallas import tpu_sc as plsc

N_NODES = 10000
TOPK = 32
L = 16
NVREG = N_NODES // L
CAPL = 32
CAP_TARGET = 144
WARM_SCALE = 0.9
ROWS_PER_W = 313
SEL_ITERS = 26
FB_MAX_ITERS = 40


def _row_count_max(row_v, thr):
    def body(i, carry):
        acc, mx = carry
        v = row_v[pl.ds(i * L, L)]
        return acc + (v > thr).astype(jnp.int32), jnp.maximum(mx, v)

    acc0 = jnp.zeros((L,), jnp.int32)
    mx0 = jnp.full((L,), -jnp.inf, jnp.float32)
    acc, mx = lax.fori_loop(0, NVREG, body, (acc0, mx0), unroll=8)
    return jnp.sum(acc), jnp.max(mx)


def _bin_pass(row_v, cand_v, cpos_v, thr, lanes, write_out, out_v):
    def body(i, c_vec):
        v = row_v[pl.ds(i * L, L)]
        m = v > thr
        if write_out:
            out_v[pl.ds(i * L, L)] = jnp.where(m, v, 0.0)
        cc = jnp.minimum(c_vec, CAPL - 1)
        idx = cc * L + lanes
        plsc.store_scatter(cand_v, [idx], v, mask=m)
        plsc.store_scatter(cpos_v, [idx], lanes + i * L, mask=m)
        return c_vec + m.astype(jnp.int32)

    return lax.fori_loop(0, NVREG, body, jnp.zeros((L,), jnp.int32),
                         unroll=8)


def _select_kth(cand_v, c_vec, mxc, lo0, n_iters):
    def hbody(j, hm):
        cj = cand_v[pl.ds(j * L, L)]
        return jnp.maximum(hm, jnp.where(c_vec > j, cj, 0.0))

    cmax = lax.fori_loop(0, mxc, hbody, jnp.zeros((L,), jnp.float32))
    hi0 = jnp.max(cmax) * 1.0001 + 1e-30

    def sel(i, carry):
        lo, hi = carry
        mid = 0.5 * (lo + hi)

        def cbody(j, acc):
            cj = cand_v[pl.ds(j * L, L)]
            keep = (c_vec > j) & (cj >= mid)
            return acc + keep.astype(jnp.int32)

        acc = lax.fori_loop(0, mxc, cbody, jnp.zeros((L,), jnp.int32))
        ge = jnp.sum(acc) >= TOPK
        return (jnp.where(ge, mid, lo), jnp.where(ge, hi, mid))

    lo, hi = lax.fori_loop(0, n_iters, sel, (lo0, hi0))
    return lo


def _sc_body(a_hbm, out_hbm, row_v, out_v, cand_v, cpos_v, sem):
    nc = 2
    wid = lax.axis_index("s") * nc + lax.axis_index("c")
    start = wid * ROWS_PER_W
    nrows = jnp.minimum(ROWS_PER_W, N_NODES - start)
    lanes = jnp.arange(L, dtype=jnp.int32)

    def row_body(r, t_prev):
        row = start + r
        pltpu.sync_copy(a_hbm.at[row], row_v)

        t1 = t_prev * WARM_SCALE

        c_vec1 = _bin_pass(row_v, cand_v, cpos_v, t1, lanes, True, out_v)
        cnt1 = jnp.sum(c_vec1)
        mxc1 = jnp.max(c_vec1)

        def fallback(_):
            npos, rowmax = _row_count_max(row_v, 0.0)

            def few_pos(_):
                return jnp.float32(0.0), jnp.zeros((L,), jnp.int32), \
                    jnp.int32(0)

            def bisect(_):
                def cond(st):
                    lo, hi, t, c, it = st
                    bad = (c < TOPK) | (c > CAP_TARGET)
                    return bad & (it < FB_MAX_ITERS)

                def step(st):
                    lo, hi, t, c, it = st
                    mid = 0.5 * (lo + hi)
                    cm, _ = _row_count_max(row_v, mid)
                    ge = cm >= TOPK
                    lo = jnp.where(ge, mid, lo)
                    hi = jnp.where(ge, hi, mid)
                    return lo, hi, mid, cm, it + 1

                lo0 = jnp.float32(0.0)
                hi0 = rowmax * 1.0001 + 1e-30
                st = lax.while_loop(
                    cond, step, (lo0, hi0, lo0, npos, jnp.int32(0)))
                lo, hi, t, c, it = st
                t = jnp.where((c < TOPK) | (c > CAP_TARGET), lo, t)
                c_vec2 = _bin_pass(row_v, cand_v, cpos_v, t, lanes,
                                   False, out_v)
                return t, c_vec2, jnp.int32(1)

            return lax.cond(npos <= TOPK, few_pos, bisect, None)

        def no_fallback(_):
            return t1, c_vec1, jnp.int32(1)

        need_fb = (cnt1 < TOPK) | (mxc1 > CAPL)
        t2, c_vec, need_select = lax.cond(need_fb, fallback, no_fallback,
                                          None)
        mxc = jnp.minimum(jnp.max(c_vec), CAPL)

        t_final = lax.cond(
            need_select != 0,
            lambda _: _select_kth(cand_v, c_vec, mxc, t2, SEL_ITERS),
            lambda _: jnp.float32(0.0), None)

        def fixup(_):
            zeros = jnp.zeros((L,), jnp.float32)

            def xbody(j, _):
                cj = cand_v[pl.ds(j * L, L)]
                pj = cpos_v[pl.ds(j * L, L)]
                mfix = (c_vec > j) & (cj < t_final)
                plsc.store_scatter(out_v, [pj], zeros, mask=mfix)
                return 0

            lax.fori_loop(0, mxc, xbody, 0)
            return jnp.int32(0)

        def rewrite(_):
            def obody(i, _):
                v = row_v[pl.ds(i * L, L)]
                out_v[pl.ds(i * L, L)] = jnp.where(v >= t_final, v, 0.0)
                return 0

            lax.fori_loop(0, NVREG, obody, 0, unroll=8)
            return jnp.int32(0)

        lax.cond(need_fb, rewrite, fixup, None)

        pltpu.sync_copy(out_v, out_hbm.at[row])
        return t_final

    lax.fori_loop(0, nrows, row_body, jnp.float32(0.0))


def _sc_topk(a):
    mesh = plsc.VectorSubcoreMesh(core_axis_name="c", subcore_axis_name="s")
    f = functools.partial(
        pl.kernel,
        mesh=mesh,
        out_type=jax.ShapeDtypeStruct((N_NODES, N_NODES), jnp.float32),
        scratch_types=[
            pltpu.VMEM((N_NODES,), jnp.float32),
            pltpu.VMEM((N_NODES,), jnp.float32),
            pltpu.VMEM((CAPL * L,), jnp.float32),
            pltpu.VMEM((CAPL * L,), jnp.int32),
            pltpu.SemaphoreType.DMA,
        ],
        compiler_params=pltpu.CompilerParams(needs_layout_passes=False),
    )(_sc_body)
    return f(a)


def kernel(idx, A_param):
    del idx
    return _sc_topk(A_param)


# speedup vs baseline: 1.1344x; 1.1344x over previous
import functools

import jax
import jax.numpy as jnp
from jax import lax
from jax.experimental import pallas as pl
from jax.experimental.pallas import tpu as pltpu
from jax.experimental.pallas import tpu_sc as plsc

N_NODES = 10000
TOPK = 32
L = 16
NVREG = N_NODES // L
CAPL = 24
CAP_TARGET = 144
WARM_SCALE = 0.9
ROWS_PER_W = 313
SEL_ITERS = 24
FB_MAX_ITERS = 40


def _row_count_max(row_v, thr):
    def body(i, carry):
        acc, mx = carry
        v = row_v[pl.ds(i * L, L)]
        return acc + (v > thr).astype(jnp.int32), jnp.maximum(mx, v)

    acc0 = jnp.zeros((L,), jnp.int32)
    mx0 = jnp.full((L,), -jnp.inf, jnp.float32)
    acc, mx = lax.fori_loop(0, NVREG, body, (acc0, mx0), unroll=8)
    return jnp.sum(acc), jnp.max(mx)


def _bin_pass(row_v, cpos_v, thr, lanes, write_out, out_v):
    def body(i, c_vec):
        v = row_v[pl.ds(i * L, L)]
        m = v > thr
        if write_out:
            out_v[pl.ds(i * L, L)] = jnp.where(m, v, 0.0)
        cc = jnp.minimum(c_vec, CAPL - 1)
        idx = cc * L + lanes
        plsc.store_scatter(cpos_v, [idx], lanes + i * L, mask=m)
        return c_vec + m.astype(jnp.int32)

    return lax.fori_loop(0, NVREG, body, jnp.zeros((L,), jnp.int32),
                         unroll=8)


def _gather_cands(row_v, cpos_v, c_vec):
    cvals = []
    for j in range(CAPL):
        pj = cpos_v[pl.ds(j * L, L)]
        cj = plsc.load_gather(row_v, [pj])
        cvals.append(jnp.where(c_vec > j, cj, 0.0))
    return cvals


def _select_kth(row_v, cpos_v, c_vec, lo0, n_iters):
    cvals = _gather_cands(row_v, cpos_v, c_vec)

    cmax = cvals[0]
    for cj in cvals[1:]:
        cmax = jnp.maximum(cmax, cj)
    hi0 = jnp.max(cmax) * 1.0001 + 1e-30

    def sel(i, carry):
        lo, hi = carry
        mid = 0.5 * (lo + hi)
        acc = jnp.zeros((L,), jnp.int32)
        for cj in cvals:
            acc = acc + (cj >= mid).astype(jnp.int32)
        ge = jnp.sum(acc) >= TOPK
        return (jnp.where(ge, mid, lo), jnp.where(ge, hi, mid))

    lo, hi = lax.fori_loop(0, n_iters, sel, (lo0, hi0))
    return lo


def _sc_body(a_hbm, out_hbm, row_v, out_v, cpos_v, sem):
    nc = 2
    wid = lax.axis_index("s") * nc + lax.axis_index("c")
    start = wid * ROWS_PER_W
    nrows = jnp.minimum(ROWS_PER_W, N_NODES - start)
    lanes = jnp.arange(L, dtype=jnp.int32)

    for j in range(CAPL):
        cpos_v[pl.ds(j * L, L)] = jnp.zeros((L,), jnp.int32)

    def row_body(r, t_prev):
        row = start + r
        pltpu.sync_copy(a_hbm.at[row], row_v)

        t1 = t_prev * WARM_SCALE

        c_vec1 = _bin_pass(row_v, cpos_v, t1, lanes, True, out_v)
        cnt1 = jnp.sum(c_vec1)
        mxc1 = jnp.max(c_vec1)

        def fallback(_):
            npos, rowmax = _row_count_max(row_v, 0.0)

            def few_pos(_):
                return jnp.float32(0.0), jnp.zeros((L,), jnp.int32), \
                    jnp.int32(0)

            def bisect(_):
                def cond(st):
                    lo, hi, t, c, it = st
                    bad = (c < TOPK) | (c > CAP_TARGET)
                    return bad & (it < FB_MAX_ITERS)

                def step(st):
                    lo, hi, t, c, it = st
                    mid = 0.5 * (lo + hi)
                    cm, _ = _row_count_max(row_v, mid)
                    ge = cm >= TOPK
                    lo = jnp.where(ge, mid, lo)
                    hi = jnp.where(ge, hi, mid)
                    return lo, hi, mid, cm, it + 1

                lo0 = jnp.float32(0.0)
                hi0 = rowmax * 1.0001 + 1e-30
                st = lax.while_loop(
                    cond, step, (lo0, hi0, lo0, npos, jnp.int32(0)))
                lo, hi, t, c, it = st
                t = jnp.where((c < TOPK) | (c > CAP_TARGET), lo, t)
                c_vec2 = _bin_pass(row_v, cpos_v, t, lanes, False, out_v)
                return t, c_vec2, jnp.int32(1)

            return lax.cond(npos <= TOPK, few_pos, bisect, None)

        def no_fallback(_):
            return t1, c_vec1, jnp.int32(1)

        need_fb = (cnt1 < TOPK) | (mxc1 > CAPL)
        t2, c_vec, need_select = lax.cond(need_fb, fallback, no_fallback,
                                          None)

        t_final = lax.cond(
            need_select != 0,
            lambda _: _select_kth(row_v, cpos_v, c_vec, t2, SEL_ITERS),
            lambda _: jnp.float32(0.0), None)

        def fixup(_):
            zeros = jnp.zeros((L,), jnp.float32)
            cvals = _gather_cands(row_v, cpos_v, c_vec)
            for j in range(CAPL):
                pj = cpos_v[pl.ds(j * L, L)]
                mfix = (c_vec > j) & (cvals[j] < t_final)
                plsc.store_scatter(out_v, [pj], zeros, mask=mfix)
            return jnp.int32(0)

        def rewrite(_):
            def obody(i, _):
                v = row_v[pl.ds(i * L, L)]
                out_v[pl.ds(i * L, L)] = jnp.where(v >= t_final, v, 0.0)
                return 0

            lax.fori_loop(0, NVREG, obody, 0, unroll=8)
            return jnp.int32(0)

        lax.cond(need_fb, rewrite, fixup, None)

        pltpu.sync_copy(out_v, out_hbm.at[row])
        return t_final

    lax.fori_loop(0, nrows, row_body, jnp.float32(0.0))


def _sc_topk(a):
    mesh = plsc.VectorSubcoreMesh(core_axis_name="c", subcore_axis_name="s")
    f = functools.partial(
        pl.kernel,
        mesh=mesh,
        out_type=jax.ShapeDtypeStruct((N_NODES, N_NODES), jnp.float32),
        scratch_types=[
            pltpu.VMEM((N_NODES,), jnp.float32),
            pltpu.VMEM((N_NODES,), jnp.float32),
            pltpu.VMEM((CAPL * L,), jnp.int32),
            pltpu.SemaphoreType.DMA,
        ],
        compiler_params=pltpu.CompilerParams(needs_layout_passes=False),
    )(_sc_body)
    return f(a)


def kernel(idx, A_param):
    del idx
    return _sc_topk(A_param)


# speedup vs baseline: 1.2707x; 1.1202x over previous
import functools

import jax
import jax.numpy as jnp
from jax import lax
from jax.experimental import pallas as pl
from jax.experimental.pallas import tpu as pltpu
from jax.experimental.pallas import tpu_sc as plsc

N_NODES = 10000
TOPK = 32
L = 16
NVREG = N_NODES // L
CAPL = 24
CAP_TARGET = 144
WARM_SCALE = 0.9
ROWS_PER_W = 313
SEL_ITERS = 24
FB_MAX_ITERS = 40


def _row_count_max(row_v, p, thr):
    def body(i, carry):
        acc, mx = carry
        v = row_v[p, pl.ds(i * L, L)]
        return acc + (v > thr).astype(jnp.int32), jnp.maximum(mx, v)

    acc0 = jnp.zeros((L,), jnp.int32)
    mx0 = jnp.full((L,), -jnp.inf, jnp.float32)
    acc, mx = lax.fori_loop(0, NVREG, body, (acc0, mx0), unroll=8)
    return jnp.sum(acc), jnp.max(mx)


def _bin_pass(row_v, cpos_v, p, thr, lanes, write_out, out_v):
    def body(i, c_vec):
        v = row_v[p, pl.ds(i * L, L)]
        m = v > thr
        if write_out:
            out_v[p, pl.ds(i * L, L)] = jnp.where(m, v, 0.0)
        cc = jnp.minimum(c_vec, CAPL - 1)
        idx = cc * L + lanes
        plsc.store_scatter(cpos_v, [idx], lanes + i * L, mask=m)
        return c_vec + m.astype(jnp.int32)

    return lax.fori_loop(0, NVREG, body, jnp.zeros((L,), jnp.int32),
                         unroll=8)


def _gather_cands(row_v, cpos_v, p_splat, c_vec):
    cvals = []
    for j in range(CAPL):
        pj = cpos_v[pl.ds(j * L, L)]
        cj = plsc.load_gather(row_v, [p_splat, pj])
        cvals.append(jnp.where(c_vec > j, cj, 0.0))
    return cvals


def _select_kth(row_v, cpos_v, p_splat, c_vec, lo0, n_iters):
    cvals = _gather_cands(row_v, cpos_v, p_splat, c_vec)

    cmax = cvals[0]
    for cj in cvals[1:]:
        cmax = jnp.maximum(cmax, cj)
    hi0 = jnp.max(cmax) * 1.0001 + 1e-30

    def sel(i, carry):
        lo, hi = carry
        mid = 0.5 * (lo + hi)
        acc = jnp.zeros((L,), jnp.int32)
        for cj in cvals:
            acc = acc + (cj >= mid).astype(jnp.int32)
        ge = jnp.sum(acc) >= TOPK
        return (jnp.where(ge, mid, lo), jnp.where(ge, hi, mid))

    lo, hi = lax.fori_loop(0, n_iters, sel, (lo0, hi0))
    return lo


def _sc_body(a_hbm, out_hbm, row_v, out_v, cpos_v, in_sem, out_sem):
    nc = 2
    wid = lax.axis_index("s") * nc + lax.axis_index("c")
    start = wid * ROWS_PER_W
    nrows = jnp.minimum(ROWS_PER_W, N_NODES - start)
    lanes = jnp.arange(L, dtype=jnp.int32)

    for j in range(CAPL):
        cpos_v[pl.ds(j * L, L)] = jnp.zeros((L,), jnp.int32)

    pltpu.async_copy(a_hbm.at[start], row_v.at[0], in_sem)

    def row_body(r, t_prev):
        row = start + r
        p = r % 2
        p_splat = jnp.full((L,), 0, jnp.int32) + p

        pltpu.make_async_copy(a_hbm.at[row], row_v.at[p], in_sem).wait()

        @pl.when(r + 1 < nrows)
        def _():
            pltpu.async_copy(a_hbm.at[row + 1], row_v.at[1 - p], in_sem)

        @pl.when(r >= 2)
        def _():
            pltpu.make_async_copy(
                out_v.at[p], out_hbm.at[row - 2], out_sem).wait()

        t1 = t_prev * WARM_SCALE

        c_vec1 = _bin_pass(row_v, cpos_v, p, t1, lanes, True, out_v)
        cnt1 = jnp.sum(c_vec1)
        mxc1 = jnp.max(c_vec1)

        def fallback(_):
            npos, rowmax = _row_count_max(row_v, p, 0.0)

            def few_pos(_):
                return jnp.float32(0.0), jnp.zeros((L,), jnp.int32), \
                    jnp.int32(0)

            def bisect(_):
                def cond(st):
                    lo, hi, t, c, it = st
                    bad = (c < TOPK) | (c > CAP_TARGET)
                    return bad & (it < FB_MAX_ITERS)

                def step(st):
                    lo, hi, t, c, it = st
                    mid = 0.5 * (lo + hi)
                    cm, _ = _row_count_max(row_v, p, mid)
                    ge = cm >= TOPK
                    lo = jnp.where(ge, mid, lo)
                    hi = jnp.where(ge, hi, mid)
                    return lo, hi, mid, cm, it + 1

                lo0 = jnp.float32(0.0)
                hi0 = rowmax * 1.0001 + 1e-30
                st = lax.while_loop(
                    cond, step, (lo0, hi0, lo0, npos, jnp.int32(0)))
                lo, hi, t, c, it = st
                t = jnp.where((c < TOPK) | (c > CAP_TARGET), lo, t)
                c_vec2 = _bin_pass(row_v, cpos_v, p, t, lanes, False,
                                   out_v)
                return t, c_vec2, jnp.int32(1)

            return lax.cond(npos <= TOPK, few_pos, bisect, None)

        def no_fallback(_):
            return t1, c_vec1, jnp.int32(1)

        need_fb = (cnt1 < TOPK) | (mxc1 > CAPL)
        t2, c_vec, need_select = lax.cond(need_fb, fallback, no_fallback,
                                          None)

        t_final = lax.cond(
            need_select != 0,
            lambda _: _select_kth(row_v, cpos_v, p_splat, c_vec, t2,
                                  SEL_ITERS),
            lambda _: jnp.float32(0.0), None)

        def fixup(_):
            zeros = jnp.zeros((L,), jnp.float32)
            cvals = _gather_cands(row_v, cpos_v, p_splat, c_vec)
            for j in range(CAPL):
                pj = cpos_v[pl.ds(j * L, L)]
                mfix = (c_vec > j) & (cvals[j] < t_final)
                plsc.store_scatter(out_v, [p_splat, pj], zeros, mask=mfix)
            return jnp.int32(0)

        def rewrite(_):
            def obody(i, _):
                v = row_v[p, pl.ds(i * L, L)]
                out_v[p, pl.ds(i * L, L)] = jnp.where(v >= t_final, v, 0.0)
                return 0

            lax.fori_loop(0, NVREG, obody, 0, unroll=8)
            return jnp.int32(0)

        lax.cond(need_fb, rewrite, fixup, None)

        pltpu.async_copy(out_v.at[p], out_hbm.at[row], out_sem)
        return t_final

    lax.fori_loop(0, nrows, row_body, jnp.float32(0.0))

    pltpu.make_async_copy(out_v.at[0], out_hbm.at[start], out_sem).wait()
    pltpu.make_async_copy(out_v.at[0], out_hbm.at[start], out_sem).wait()


def _sc_topk(a):
    mesh = plsc.VectorSubcoreMesh(core_axis_name="c", subcore_axis_name="s")
    f = functools.partial(
        pl.kernel,
        mesh=mesh,
        out_type=jax.ShapeDtypeStruct((N_NODES, N_NODES), jnp.float32),
        scratch_types=[
            pltpu.VMEM((2, N_NODES), jnp.float32),
            pltpu.VMEM((2, N_NODES), jnp.float32),
            pltpu.VMEM((CAPL * L,), jnp.int32),
            pltpu.SemaphoreType.DMA,
            pltpu.SemaphoreType.DMA,
        ],
        compiler_params=pltpu.CompilerParams(needs_layout_passes=False),
    )(_sc_body)
    return f(a)


def kernel(idx, A_param):
    del idx
    return _sc_topk(A_param)


# speedup vs baseline: 1.2755x; 1.0037x over previous
import functools

import jax
import jax.numpy as jnp
from jax import lax
from jax.experimental import pallas as pl
from jax.experimental.pallas import tpu as pltpu
from jax.experimental.pallas import tpu_sc as plsc

N_NODES = 10000
TOPK = 32
L = 16
NVREG = N_NODES // L
CAPL = 24
CAP_TARGET = 144
WARM_SCALE = 0.85
ROWS_PER_W = 313
SEL_ITERS = 24
FB_MAX_ITERS = 40


def _row_count_max(row_v, p, thr):
    def body(i, carry):
        acc, mx = carry
        v = row_v[p, pl.ds(i * L, L)]
        return acc + (v > thr).astype(jnp.int32), jnp.maximum(mx, v)

    acc0 = jnp.zeros((L,), jnp.int32)
    mx0 = jnp.full((L,), -jnp.inf, jnp.float32)
    acc, mx = lax.fori_loop(0, NVREG, body, (acc0, mx0), unroll=8)
    return jnp.sum(acc), jnp.max(mx)


def _bin_pass(row_v, cpos_v, p, thr, lanes, write_out, out_v):
    def body(i, c_vec):
        v = row_v[p, pl.ds(i * L, L)]
        m = v > thr
        if write_out:
            out_v[p, pl.ds(i * L, L)] = jnp.where(m, v, 0.0)
        cc = jnp.minimum(c_vec, CAPL - 1)
        idx = cc * L + lanes
        plsc.store_scatter(cpos_v, [idx], lanes + i * L, mask=m)
        return c_vec + m.astype(jnp.int32)

    return lax.fori_loop(0, NVREG, body, jnp.zeros((L,), jnp.int32),
                         unroll=16)


def _gather_cands(row_v, cpos_v, p_splat, c_vec):
    cvals = []
    for j in range(CAPL):
        pj = cpos_v[pl.ds(j * L, L)]
        cj = plsc.load_gather(row_v, [p_splat, pj])
        cvals.append(jnp.where(c_vec > j, cj, 0.0))
    return cvals


def _select_kth(row_v, cpos_v, p_splat, c_vec, lo0, n_iters):
    cvals = _gather_cands(row_v, cpos_v, p_splat, c_vec)

    cmax = cvals[0]
    for cj in cvals[1:]:
        cmax = jnp.maximum(cmax, cj)
    hi0 = jnp.max(cmax) * 1.0001 + 1e-30

    def sel(i, carry):
        lo, hi = carry
        mid = 0.5 * (lo + hi)
        acc = jnp.zeros((L,), jnp.int32)
        for cj in cvals:
            acc = acc + (cj >= mid).astype(jnp.int32)
        ge = jnp.sum(acc) >= TOPK
        return (jnp.where(ge, mid, lo), jnp.where(ge, hi, mid))

    lo, hi = lax.fori_loop(0, n_iters, sel, (lo0, hi0))
    return lo


def _sc_body(a_hbm, out_hbm, row_v, out_v, cpos_v, in_sem, out_sem):
    nc = 2
    wid = lax.axis_index("s") * nc + lax.axis_index("c")
    start = wid * ROWS_PER_W
    nrows = jnp.minimum(ROWS_PER_W, N_NODES - start)
    lanes = jnp.arange(L, dtype=jnp.int32)

    for j in range(CAPL):
        cpos_v[pl.ds(j * L, L)] = jnp.zeros((L,), jnp.int32)

    pltpu.async_copy(a_hbm.at[start], row_v.at[0], in_sem)

    def row_body(r, t_prev):
        row = start + r
        p = r % 2
        p_splat = jnp.full((L,), 0, jnp.int32) + p

        pltpu.make_async_copy(a_hbm.at[row], row_v.at[p], in_sem).wait()

        @pl.when(r + 1 < nrows)
        def _():
            pltpu.async_copy(a_hbm.at[row + 1], row_v.at[1 - p], in_sem)

        @pl.when(r >= 2)
        def _():
            pltpu.make_async_copy(
                out_v.at[p], out_hbm.at[row - 2], out_sem).wait()

        t1 = t_prev * WARM_SCALE

        c_vec1 = _bin_pass(row_v, cpos_v, p, t1, lanes, True, out_v)
        cnt1 = jnp.sum(c_vec1)
        mxc1 = jnp.max(c_vec1)

        def fallback(_):
            npos, rowmax = _row_count_max(row_v, p, 0.0)

            def few_pos(_):
                return jnp.float32(0.0), jnp.zeros((L,), jnp.int32), \
                    jnp.int32(0)

            def bisect(_):
                def cond(st):
                    lo, hi, t, c, it = st
                    bad = (c < TOPK) | (c > CAP_TARGET)
                    return bad & (it < FB_MAX_ITERS)

                def step(st):
                    lo, hi, t, c, it = st
                    mid = 0.5 * (lo + hi)
                    cm, _ = _row_count_max(row_v, p, mid)
                    ge = cm >= TOPK
                    lo = jnp.where(ge, mid, lo)
                    hi = jnp.where(ge, hi, mid)
                    return lo, hi, mid, cm, it + 1

                lo0 = jnp.float32(0.0)
                hi0 = rowmax * 1.0001 + 1e-30
                st = lax.while_loop(
                    cond, step, (lo0, hi0, lo0, npos, jnp.int32(0)))
                lo, hi, t, c, it = st
                t = jnp.where((c < TOPK) | (c > CAP_TARGET), lo, t)
                c_vec2 = _bin_pass(row_v, cpos_v, p, t, lanes, False,
                                   out_v)
                return t, c_vec2, jnp.int32(1)

            return lax.cond(npos <= TOPK, few_pos, bisect, None)

        def no_fallback(_):
            return t1, c_vec1, jnp.int32(1)

        need_fb = (cnt1 < TOPK) | (mxc1 > CAPL)
        t2, c_vec, need_select = lax.cond(need_fb, fallback, no_fallback,
                                          None)

        t_final = lax.cond(
            need_select != 0,
            lambda _: _select_kth(row_v, cpos_v, p_splat, c_vec, t2,
                                  SEL_ITERS),
            lambda _: jnp.float32(0.0), None)

        def fixup(_):
            zeros = jnp.zeros((L,), jnp.float32)
            cvals = _gather_cands(row_v, cpos_v, p_splat, c_vec)
            for j in range(CAPL):
                pj = cpos_v[pl.ds(j * L, L)]
                mfix = (c_vec > j) & (cvals[j] < t_final)
                plsc.store_scatter(out_v, [p_splat, pj], zeros, mask=mfix)
            return jnp.int32(0)

        def rewrite(_):
            def obody(i, _):
                v = row_v[p, pl.ds(i * L, L)]
                out_v[p, pl.ds(i * L, L)] = jnp.where(v >= t_final, v, 0.0)
                return 0

            lax.fori_loop(0, NVREG, obody, 0, unroll=8)
            return jnp.int32(0)

        lax.cond(need_fb, rewrite, fixup, None)

        pltpu.async_copy(out_v.at[p], out_hbm.at[row], out_sem)
        return t_final

    lax.fori_loop(0, nrows, row_body, jnp.float32(0.0))

    pltpu.make_async_copy(out_v.at[0], out_hbm.at[start], out_sem).wait()
    pltpu.make_async_copy(out_v.at[0], out_hbm.at[start], out_sem).wait()


def _sc_topk(a):
    mesh = plsc.VectorSubcoreMesh(core_axis_name="c", subcore_axis_name="s")
    f = functools.partial(
        pl.kernel,
        mesh=mesh,
        out_type=jax.ShapeDtypeStruct((N_NODES, N_NODES), jnp.float32),
        scratch_types=[
            pltpu.VMEM((2, N_NODES), jnp.float32),
            pltpu.VMEM((2, N_NODES), jnp.float32),
            pltpu.VMEM((CAPL * L,), jnp.int32),
            pltpu.SemaphoreType.DMA,
            pltpu.SemaphoreType.DMA,
        ],
        compiler_params=pltpu.CompilerParams(needs_layout_passes=False),
    )(_sc_body)
    return f(a)


def kernel(idx, A_param):
    del idx
    return _sc_topk(A_param)
